# baseline (speedup 1.0000x reference)
import functools

import jax
import jax.numpy as jnp
from jax import lax
from jax.experimental import pallas as pl
from jax.experimental.pallas import tpu as pltpu
from jax.experimental.pallas import tpu_sc as plsc

F = 100
C = 1000
D = 32
B = 16384
LANES = 16

BT = 128
N_BT = B // BT
G = 4
GB = G * BT


def _make_kernel():
  mesh = plsc.VectorSubcoreMesh(core_axis_name="c", subcore_axis_name="s")
  nw = mesh.num_cores * mesh.num_subcores
  assert N_BT == nw * G

  def body(xt_hbm, tbl_hbm, out_hbm, idx_all, rows0, rows1, trans0, trans1,
           gsem0, gsem1, osem0, osem1):
    wid = lax.axis_index("s") * mesh.num_cores + lax.axis_index("c")
    ri = lax.iota(jnp.int32, LANES)
    cr = []
    for bt in range(G):
      for dh in (0, LANES):
        dv = ri + dh
        cr.append(((dv >> 3) << 5) + (bt * 8) + (dv & 7))

    pltpu.sync_copy(xt_hbm.at[:, pl.ds(wid * G, G), :], idx_all)

    @pl.loop(0, F)
    def adjust(f):
      off = f * C
      for j in range(G):
        for i in range(BT // LANES):
          sl = idx_all.at[f, j, pl.ds(i * LANES, LANES)]
          idx_all[f, j, pl.ds(i * LANES, LANES)] = sl[...] + off

    def fire_gathers(f, rows, gsem):
      for j in range(G):
        pltpu.async_copy(
            tbl_hbm.at[idx_all.at[f, j]],
            rows.at[pl.ds(j * BT, BT), pl.ds(0, D)],
            gsem,
        )

    def drain_gathers(rows, gsem):
      for j in range(G):
        pltpu.make_async_copy(
            tbl_hbm.at[idx_all.at[0, j]],
            rows.at[pl.ds(j * BT, BT), pl.ds(0, D)],
            gsem,
        ).wait()

    def transpose(rows, trans):
      for bt in range(G):
        c0, c1 = cr[2 * bt], cr[2 * bt + 1]

        @plsc.parallel_loop(0, BT, unroll=8)
        def rowloop(l):
          k = bt * BT + l
          lv = jnp.zeros((LANES,), jnp.int32) + l
          plsc.store_scatter(trans, [c0, lv], rows[k, pl.ds(0, LANES)])
          plsc.store_scatter(trans, [c1, lv], rows[k, pl.ds(LANES, LANES)])

    def fire_out(f, trans, osem):
      for dt in range(D // 8):
        pltpu.async_copy(
            trans.at[pl.ds(dt * 32, 32), pl.ds(0, BT)],
            out_hbm.at[f, dt, pl.ds(wid * G * 8, 32)],
            osem,
        )

    def drain_out(trans, osem):
      for dt in range(D // 8):
        pltpu.make_async_copy(
            trans.at[pl.ds(dt * 32, 32), pl.ds(0, BT)],
            out_hbm.at[0, dt, pl.ds(wid * G * 8, 32)],
            osem,
        ).wait()

    fire_gathers(0, rows0, gsem0)

    @pl.loop(0, F, step=2)
    def floop(f):
      fire_gathers(f + 1, rows1, gsem1)
      drain_gathers(rows0, gsem0)

      @pl.when(f > 0)
      def _():
        drain_out(trans0, osem0)

      transpose(rows0, trans0)
      fire_out(f, trans0, osem0)

      @pl.when(f + 2 < F)
      def _():
        fire_gathers(f + 2, rows0, gsem0)

      drain_gathers(rows1, gsem1)

      @pl.when(f > 0)
      def _():
        drain_out(trans1, osem1)

      transpose(rows1, trans1)
      fire_out(f + 1, trans1, osem1)

    drain_out(trans0, osem0)
    drain_out(trans1, osem1)

  return pl.kernel(
      body,
      out_type=jax.ShapeDtypeStruct((F, D // 8, N_BT * 8, BT), jnp.float32),
      mesh=mesh,
      scratch_types=[
          pltpu.VMEM((F, G, BT), jnp.int32),
          pltpu.VMEM((GB, D), jnp.float32),
          pltpu.VMEM((GB, D), jnp.float32),
          pltpu.VMEM((D // 8 * G * 8, BT + 1), jnp.float32),
          pltpu.VMEM((D // 8 * G * 8, BT + 1), jnp.float32),
          pltpu.SemaphoreType.DMA,
          pltpu.SemaphoreType.DMA,
          pltpu.SemaphoreType.DMA,
          pltpu.SemaphoreType.DMA,
      ],
      compiler_params=pltpu.CompilerParams(
          use_tc_tiling_on_sc=False, needs_layout_passes=False
      ),
  )


def kernel(x, weight):
  xt = x.astype(jnp.int32).T.reshape(F, N_BT, BT)
  tbl = weight.reshape(F * C, D)
  u = _make_kernel()(xt, tbl).reshape(F, D // 8, N_BT, 8, BT)
  return jnp.transpose(u, (2, 4, 0, 1, 3)).reshape(B, F, D)


# speedup vs baseline: 1.0045x; 1.0045x over previous
import functools

import jax
import jax.numpy as jnp
from jax import lax
from jax.experimental import pallas as pl
from jax.experimental.pallas import tpu as pltpu
from jax.experimental.pallas import tpu_sc as plsc

F = 100
C = 1000
D = 32
B = 16384
LANES = 16

BT = 128
N_BT = B // BT
G = 4
GB = G * BT


def _make_kernel():
  mesh = plsc.VectorSubcoreMesh(core_axis_name="c", subcore_axis_name="s")
  nw = mesh.num_cores * mesh.num_subcores
  assert N_BT == nw * G

  def body(xt_hbm, tbl_hbm, out_hbm, idx_all, rows0, rows1, trans0, trans1,
           gsem0, gsem1, osem0, osem1):
    wid = lax.axis_index("s") * mesh.num_cores + lax.axis_index("c")
    ri = lax.iota(jnp.int32, LANES)
    cr = []
    for bt in range(G):
      for dh in (0, LANES):
        dv = ri + dh
        cr.append(((dv >> 3) << 5) + (bt * 8) + (dv & 7))

    pltpu.sync_copy(xt_hbm.at[:, pl.ds(wid * G, G), :], idx_all)

    def adjust(f):
      off = f * C

      @plsc.parallel_loop(0, G * (BT // LANES), unroll=8)
      def adj(k):
        j = k >> 3
        i = k & 7
        sl = idx_all.at[f, j, pl.ds(i * LANES, LANES)]
        idx_all[f, j, pl.ds(i * LANES, LANES)] = sl[...] + off

    def fire_gathers(f, rows, gsem):
      for j in range(G):
        pltpu.async_copy(
            tbl_hbm.at[idx_all.at[f, j]],
            rows.at[pl.ds(j * BT, BT), pl.ds(0, D)],
            gsem,
        )

    def drain_gathers(rows, gsem):
      for j in range(G):
        pltpu.make_async_copy(
            tbl_hbm.at[idx_all.at[0, j]],
            rows.at[pl.ds(j * BT, BT), pl.ds(0, D)],
            gsem,
        ).wait()

    def transpose(rows, trans):
      for bt in range(G):
        c0, c1 = cr[2 * bt], cr[2 * bt + 1]

        @plsc.parallel_loop(0, BT, unroll=8)
        def rowloop(l):
          k = bt * BT + l
          lv = jnp.zeros((LANES,), jnp.int32) + l
          plsc.store_scatter(trans, [c0, lv], rows[k, pl.ds(0, LANES)])
          plsc.store_scatter(trans, [c1, lv], rows[k, pl.ds(LANES, LANES)])

    def fire_out(f, trans, osem):
      for dt in range(D // 8):
        pltpu.async_copy(
            trans.at[pl.ds(dt * 32, 32), pl.ds(0, BT)],
            out_hbm.at[f, dt, pl.ds(wid * G * 8, 32)],
            osem,
        )

    def drain_out(trans, osem):
      for dt in range(D // 8):
        pltpu.make_async_copy(
            trans.at[pl.ds(dt * 32, 32), pl.ds(0, BT)],
            out_hbm.at[0, dt, pl.ds(wid * G * 8, 32)],
            osem,
        ).wait()

    adjust(0)
    fire_gathers(0, rows0, gsem0)

    @pl.loop(0, F, step=2)
    def floop(f):
      adjust(f + 1)
      fire_gathers(f + 1, rows1, gsem1)
      drain_gathers(rows0, gsem0)

      @pl.when(f > 0)
      def _():
        drain_out(trans0, osem0)

      transpose(rows0, trans0)
      fire_out(f, trans0, osem0)

      @pl.when(f + 2 < F)
      def _():
        adjust(f + 2)
        fire_gathers(f + 2, rows0, gsem0)

      drain_gathers(rows1, gsem1)

      @pl.when(f > 0)
      def _():
        drain_out(trans1, osem1)

      transpose(rows1, trans1)
      fire_out(f + 1, trans1, osem1)

    drain_out(trans0, osem0)
    drain_out(trans1, osem1)

  return pl.kernel(
      body,
      out_type=jax.ShapeDtypeStruct((F, D // 8, N_BT * 8, BT), jnp.float32),
      mesh=mesh,
      scratch_types=[
          pltpu.VMEM((F, G, BT), jnp.int32),
          pltpu.VMEM((GB, D), jnp.float32),
          pltpu.VMEM((GB, D), jnp.float32),
          pltpu.VMEM((D // 8 * G * 8, BT + 1), jnp.float32),
          pltpu.VMEM((D // 8 * G * 8, BT + 1), jnp.float32),
          pltpu.SemaphoreType.DMA,
          pltpu.SemaphoreType.DMA,
          pltpu.SemaphoreType.DMA,
          pltpu.SemaphoreType.DMA,
      ],
      compiler_params=pltpu.CompilerParams(
          use_tc_tiling_on_sc=False, needs_layout_passes=False
      ),
  )


def kernel(x, weight):
  xt = x.astype(jnp.int32).T.reshape(F, N_BT, BT)
  tbl = weight.reshape(F * C, D)
  u = _make_kernel()(xt, tbl).reshape(F, D // 8, N_BT, 8, BT)
  return jnp.transpose(u, (2, 4, 0, 1, 3)).reshape(B, F, D)


# speedup vs baseline: 1.0068x; 1.0023x over previous
import jax
import jax.numpy as jnp
from jax import lax
from jax.experimental import pallas as pl
from jax.experimental.pallas import tpu as pltpu
from jax.experimental.pallas import tpu_sc as plsc

F = 100
C = 1000
D = 32
B = 16384
LANES = 16

BT = 128
N_BT = B // BT
G = 4
GB = G * BT


def _make_kernel():
  mesh = plsc.VectorSubcoreMesh(core_axis_name="c", subcore_axis_name="s")
  nw = mesh.num_cores * mesh.num_subcores
  assert N_BT == nw * G

  def body(xt_hbm, tbl_hbm, out_hbm, idx_all, rows0, rows1, trans0, trans1,
           gsem0, gsem1, osem0, osem1):
    wid = lax.axis_index("s") * mesh.num_cores + lax.axis_index("c")
    ri = lax.iota(jnp.int32, LANES)
    cr = []
    for bt in range(G):
      for dh in (0, LANES):
        dv = ri + dh
        cr.append(((dv >> 3) << 5) + (bt * 8) + (dv & 7))

    pltpu.sync_copy(xt_hbm.at[:, pl.ds(wid * G, G), :], idx_all)

    def adjust(f):
      off = f * C

      @plsc.parallel_loop(0, G * (BT // LANES), unroll=8)
      def adj(k):
        j = k >> 3
        i = k & 7
        sl = idx_all.at[f, j, pl.ds(i * LANES, LANES)]
        idx_all[f, j, pl.ds(i * LANES, LANES)] = sl[...] + off

    def fire_gathers(f, rows, gsem):
      for j in range(G):
        pltpu.async_copy(
            tbl_hbm.at[idx_all.at[f, j]],
            rows.at[pl.ds(j * BT, BT), pl.ds(0, D)],
            gsem,
        )

    def drain_gathers(rows, gsem):
      for j in range(G):
        pltpu.make_async_copy(
            tbl_hbm.at[idx_all.at[0, j]],
            rows.at[pl.ds(j * BT, BT), pl.ds(0, D)],
            gsem,
        ).wait()

    def transpose(rows, trans):
      for bt in range(G):
        c0, c1 = cr[2 * bt], cr[2 * bt + 1]

        @plsc.parallel_loop(0, BT, unroll=8)
        def rowloop(l):
          k = bt * BT + l
          lv = jnp.zeros((LANES,), jnp.int32) + l
          plsc.store_scatter(trans, [c0, lv], rows[k, pl.ds(0, LANES)])
          plsc.store_scatter(trans, [c1, lv], rows[k, pl.ds(LANES, LANES)])

    def fire_out(f, trans, osem):
      for dt in range(D // 8):
        pltpu.async_copy(
            trans.at[pl.ds(dt * 32, 32), pl.ds(0, BT)],
            out_hbm.at[f, dt, pl.ds(wid * G * 8, 32)],
            osem,
        )

    def drain_out(trans, osem):
      for dt in range(D // 8):
        pltpu.make_async_copy(
            trans.at[pl.ds(dt * 32, 32), pl.ds(0, BT)],
            out_hbm.at[0, dt, pl.ds(wid * G * 8, 32)],
            osem,
        ).wait()

    adjust(0)
    fire_gathers(0, rows0, gsem0)

    @pl.loop(0, F, step=2)
    def floop(f):
      adjust(f + 1)
      fire_gathers(f + 1, rows1, gsem1)
      drain_gathers(rows0, gsem0)

      @pl.when(f > 0)
      def _():
        drain_out(trans0, osem0)

      transpose(rows0, trans0)
      fire_out(f, trans0, osem0)

      @pl.when(f + 2 < F)
      def _():
        adjust(f + 2)
        fire_gathers(f + 2, rows0, gsem0)

      drain_gathers(rows1, gsem1)

      @pl.when(f > 0)
      def _():
        drain_out(trans1, osem1)

      transpose(rows1, trans1)
      fire_out(f + 1, trans1, osem1)

    drain_out(trans0, osem0)
    drain_out(trans1, osem1)

  return pl.kernel(
      body,
      out_type=jax.ShapeDtypeStruct((F, D // 8, N_BT * 8, BT), jnp.float32),
      mesh=mesh,
      scratch_types=[
          pltpu.VMEM((F, G, BT), jnp.int32),
          pltpu.VMEM((GB, D), jnp.float32),
          pltpu.VMEM((GB, D), jnp.float32),
          pltpu.VMEM((D // 8 * G * 8, BT + 1), jnp.float32),
          pltpu.VMEM((D // 8 * G * 8, BT + 1), jnp.float32),
          pltpu.SemaphoreType.DMA,
          pltpu.SemaphoreType.DMA,
          pltpu.SemaphoreType.DMA,
          pltpu.SemaphoreType.DMA,
      ],
      compiler_params=pltpu.CompilerParams(
          use_tc_tiling_on_sc=False, needs_layout_passes=False
      ),
  )


def kernel(x, weight):
  xt = x.astype(jnp.int32).T.reshape(F, N_BT, BT)
  tbl = weight.reshape(F * C, D)
  u = _make_kernel()(xt, tbl).reshape(F, D // 8, N_BT, 8, BT)
  return jnp.transpose(u, (2, 4, 0, 1, 3)).reshape(B, F, D)
